# baseline (speedup 1.0000x reference)
import functools

import jax
import jax.numpy as jnp
from jax import lax
from jax.experimental import pallas as pl
from jax.experimental.pallas import tpu as pltpu
from jax.experimental.pallas import tpu_sc as plsc

D = 64
L = 16
SCALE = 8.0
NC = 2
NS = 16
NW = NC * NS
IDXW = 128
CHUNK = 512


def _make_sc_kernel(B):
    b_per_w = B // NW
    n_chunks = b_per_w // CHUNK
    n_streams = CHUNK // IDXW
    idx_rows_per_w = b_per_w // IDXW
    mesh = plsc.VectorSubcoreMesh(core_axis_name="c", subcore_axis_name="s")

    @functools.partial(
        pl.kernel,
        out_type=jax.ShapeDtypeStruct((B, D), jnp.float32),
        mesh=mesh,
        scratch_types=[
            pltpu.VMEM((n_streams, IDXW), jnp.int32),
            pltpu.VMEM((CHUNK, D), jnp.float32),
            pltpu.SemaphoreType.DMA,
        ],
        compiler_params=pltpu.CompilerParams(use_tc_tiling_on_sc=False),
    )
    def k(idx_hbm, lut_hbm, out_hbm, idx_v, rows_v, sem):
        wid = lax.axis_index("s") * NC + lax.axis_index("c")
        base = wid * b_per_w
        base_row = wid * idx_rows_per_w

        def chunk_body(c, carry):
            off = base + c * CHUNK
            pltpu.sync_copy(
                idx_hbm.at[pl.ds(base_row + c * n_streams, n_streams)],
                idx_v.at[...],
            )
            copies = []
            for j in range(n_streams):
                copies.append(
                    pltpu.async_copy(
                        lut_hbm.at[idx_v.at[j]],
                        rows_v.at[pl.ds(j * IDXW, IDXW)],
                        sem,
                    )
                )
            for cp in copies:
                cp.wait()

            def scale_body(i, carry2):
                row = i * 4
                for r in range(4):
                    for jj in range(D // L):
                        sl = (row + r, pl.ds(jj * L, L))
                        rows_v[sl] = rows_v[sl] * SCALE
                return carry2

            lax.fori_loop(0, CHUNK // 4, scale_body, 0)
            pltpu.sync_copy(rows_v, out_hbm.at[pl.ds(off, CHUNK)])
            return carry

        lax.fori_loop(0, n_chunks, chunk_body, 0)

    return k


def kernel(x, lut):
    S, T = x.shape
    B = S * T
    xf = x.reshape(B // IDXW, IDXW)
    out = _make_sc_kernel(B)(xf, lut)
    return out.reshape(S, T, D)


# speedup vs baseline: 1.0764x; 1.0764x over previous
import functools

import jax
import jax.numpy as jnp
from jax import lax
from jax.experimental import pallas as pl
from jax.experimental.pallas import tpu as pltpu
from jax.experimental.pallas import tpu_sc as plsc

D = 64
L = 16
SCALE = 8.0
NC = 2
NS = 16
NW = NC * NS
IDXW = 128
CHUNK = 512
NSTR = CHUNK // IDXW


def _make_sc_kernel(B):
    b_per_w = B // NW
    n_chunks = b_per_w // CHUNK
    idx_rows_per_w = b_per_w // IDXW
    mesh = plsc.VectorSubcoreMesh(core_axis_name="c", subcore_axis_name="s")

    @functools.partial(
        pl.kernel,
        out_type=jax.ShapeDtypeStruct((B, D), jnp.float32),
        mesh=mesh,
        scratch_types=[
            pltpu.VMEM((NSTR, IDXW), jnp.int32),
            pltpu.VMEM((NSTR, IDXW), jnp.int32),
            pltpu.VMEM((CHUNK, D), jnp.float32),
            pltpu.VMEM((CHUNK, D), jnp.float32),
            pltpu.SemaphoreType.DMA,
            pltpu.SemaphoreType.DMA,
            pltpu.SemaphoreType.DMA,
            pltpu.SemaphoreType.DMA,
        ],
        compiler_params=pltpu.CompilerParams(use_tc_tiling_on_sc=False),
    )
    def k(idx_hbm, lut_hbm, out_hbm, idx0, idx1, rows0, rows1,
          gsem0, gsem1, osem0, osem1):
        idx_v = (idx0, idx1)
        rows_v = (rows0, rows1)
        gsem = (gsem0, gsem1)
        osem = (osem0, osem1)
        wid = lax.axis_index("s") * NC + lax.axis_index("c")
        base = wid * b_per_w
        base_row = wid * idx_rows_per_w

        def fire_gather(c, b):
            pltpu.sync_copy(
                idx_hbm.at[pl.ds(base_row + c * NSTR, NSTR)],
                idx_v[b].at[...],
            )
            for j in range(NSTR):
                pltpu.async_copy(
                    lut_hbm.at[idx_v[b].at[j]],
                    rows_v[b].at[pl.ds(j * IDXW, IDXW)],
                    gsem[b],
                )

        def drain_gather(b):
            for j in range(NSTR):
                pltpu.make_async_copy(
                    lut_hbm.at[idx_v[b].at[j]],
                    rows_v[b].at[pl.ds(j * IDXW, IDXW)],
                    gsem[b],
                ).wait()

        def scale(b):
            def body(i, carry):
                row = i * 4
                for r in range(4):
                    for jj in range(D // L):
                        sl = (row + r, pl.ds(jj * L, L))
                        rows_v[b][sl] = rows_v[b][sl] * SCALE
                return carry
            lax.fori_loop(0, CHUNK // 4, body, 0)

        fire_gather(0, 0)

        def step(c, b):
            nb = 1 - b

            @pl.when(c >= 1)
            def _wait_prev_scatter():
                pltpu.make_async_copy(
                    rows_v[nb],
                    out_hbm.at[pl.ds(base, CHUNK)],
                    osem[nb],
                ).wait()

            @pl.when(c + 1 < n_chunks)
            def _prefetch():
                fire_gather(c + 1, nb)

            drain_gather(b)
            scale(b)
            pltpu.async_copy(
                rows_v[b],
                out_hbm.at[pl.ds(base + c * CHUNK, CHUNK)],
                osem[b],
            )

        def outer(c2, carry):
            step(c2 * 2, 0)
            step(c2 * 2 + 1, 1)
            return carry

        lax.fori_loop(0, n_chunks // 2, outer, 0)

        pltpu.make_async_copy(
            rows_v[(n_chunks - 1) % 2],
            out_hbm.at[pl.ds(base, CHUNK)],
            osem[(n_chunks - 1) % 2],
        ).wait()

    return k


def kernel(x, lut):
    S, T = x.shape
    B = S * T
    xf = x.reshape(B // IDXW, IDXW)
    out = _make_sc_kernel(B)(xf, lut)
    return out.reshape(S, T, D)
